# baseline (speedup 1.0000x reference)
import functools

import jax
import jax.numpy as jnp
from jax import lax
from jax.experimental import pallas as pl
from jax.experimental.pallas import tpu as pltpu
from jax.experimental.pallas import tpu_sc as plsc

NUM_BUCKETS = 1000000
EMBED_DIM = 64
LANES = 16
BATCH = 16384
SEQ = 50

PAIR_ROWS = 500032
FULL_TILES = 7812
MAIN_TILES = 7808
TILES_PER_W = 244
UNITS_PER_W = 200


def _mesh():
    return plsc.VectorSubcoreMesh(core_axis_name="c", subcore_axis_name="s")


def _wid(nc):
    return lax.axis_index("s") * nc + lax.axis_index("c")


@functools.lru_cache(maxsize=None)
def _make_repack():
    info = plsc.get_sparse_core_info()
    nc = info.num_cores

    @functools.partial(
        pl.kernel,
        mesh=_mesh(),
        out_type=jax.ShapeDtypeStruct((PAIR_ROWS, 2 * EMBED_DIM), jnp.float32),
        compiler_params=pltpu.CompilerParams(
            use_tc_tiling_on_sc=True,
            needs_layout_passes=False,
            disable_bounds_checks=True,
        ),
        scratch_types=[
            [pltpu.VMEM((EMBED_DIM, 128), jnp.float32) for _ in range(2)],
            [pltpu.VMEM((EMBED_DIM, 2 * EMBED_DIM), jnp.float32) for _ in range(2)],
            [pltpu.SemaphoreType.DMA for _ in range(2)],
            [pltpu.SemaphoreType.DMA for _ in range(2)],
        ],
    )
    def repack(table_t, tail, scr, bin_, bouts, rsem, wsem):
        wid = _wid(nc)
        iota = lax.iota(jnp.int32, LANES)

        def read(ti, b):
            pltpu.async_copy(
                table_t.at[:, pl.ds(ti * 128, 128)], bin_[b], rsem[b]
            )

        def wait_read(b):
            pltpu.make_async_copy(
                table_t.at[:, pl.ds(0, 128)], bin_[b], rsem[b]
            ).wait()

        def transpose(b, n_buckets):
            bout = bouts[b]
            n_blocks = 4 * (n_buckets // LANES)

            @plsc.parallel_loop(0, n_blocks, unroll=4)
            def blk(bi):
                dgi = lax.shift_right_logical(bi, 3)
                jbi = bi & 7
                dsrc = iota + dgi * LANES
                j0 = jbi * LANES
                for t in range(LANES):
                    perm = (iota + t) & 15
                    jsrc = perm + j0
                    val = plsc.load_gather(bin_[b], [dsrc, jsrc])
                    rowdst = lax.shift_right_logical(jsrc, 1)
                    coldst = (jsrc & 1) * EMBED_DIM + dsrc
                    plsc.store_scatter(bout, [rowdst, coldst], val)

        def write(ti, b):
            pltpu.async_copy(bouts[b], scr.at[pl.ds(ti * 64, 64), :], wsem[b])

        def wait_write(b):
            pltpu.make_async_copy(
                bouts[b], scr.at[pl.ds(0, 64), :], wsem[b]
            ).wait()

        read(wid, 0)

        def pair_body(k2, carry):
            for b in range(2):
                k = k2 * 2 + b
                ti = wid + k * 32

                @pl.when(k + 1 < TILES_PER_W)
                def _():
                    read(wid + (k + 1) * 32, 1 - b)

                wait_read(b)

                @pl.when(k >= 2)
                def _():
                    wait_write(b)

                transpose(b, 128)
                write(ti, b)
            return carry

        lax.fori_loop(0, TILES_PER_W // 2, pair_body, 0)
        wait_write(0)
        wait_write(1)

        @pl.when(wid < FULL_TILES - MAIN_TILES)
        def _():
            ti = MAIN_TILES + wid
            pltpu.sync_copy(table_t.at[:, pl.ds(ti * 128, 128)], bin_[0])
            transpose(0, 128)
            write(ti, 0)
            wait_write(0)

        @pl.when(wid == 4)
        def _():
            pltpu.sync_copy(tail, scr.at[pl.ds(FULL_TILES * 64, 32), :])

    return repack


@functools.lru_cache(maxsize=None)
def _make_lookup():
    info = plsc.get_sparse_core_info()
    nc = info.num_cores
    n_units = UNITS_PER_W

    @functools.partial(
        pl.kernel,
        mesh=_mesh(),
        out_type=jax.ShapeDtypeStruct((SEQ, EMBED_DIM, BATCH), jnp.float32),
        compiler_params=pltpu.CompilerParams(
            use_tc_tiling_on_sc=True,
            needs_layout_passes=False,
            disable_bounds_checks=True,
        ),
        scratch_types=[
            [pltpu.VMEM((128,), jnp.int32) for _ in range(4)],
            [pltpu.VMEM((128,), jnp.int32) for _ in range(4)],
            [pltpu.VMEM((128,), jnp.int32) for _ in range(4)],
            [pltpu.VMEM((128, 2 * EMBED_DIM), jnp.float32) for _ in range(4)],
            [pltpu.VMEM((EMBED_DIM, 128), jnp.float32) for _ in range(2)],
            [pltpu.SemaphoreType.DMA for _ in range(4)],
            [pltpu.SemaphoreType.DMA for _ in range(4)],
            [pltpu.SemaphoreType.DMA for _ in range(2)],
        ],
    )
    def lookup(scr, fv_t, out, ibuf, hbuf, pbuf, gbuf, obuf, isem, gsem, osem):
        wid = _wid(nc)
        u0 = wid * n_units
        iota = lax.iota(jnp.int32, LANES)

        def idx_load(k, b):
            u = u0 + k
            s = lax.shift_right_logical(u, 7)
            tb = u & 127
            pltpu.async_copy(
                fv_t.at[s, pl.ds(tb * 128, 128)], ibuf[b], isem[b]
            )

        def idx_wait(b):
            pltpu.make_async_copy(
                fv_t.at[0, pl.ds(0, 128)], ibuf[b], isem[b]
            ).wait()

        def mod(b):
            for i in range(8):
                h = ibuf[b][pl.ds(i * LANES, LANES)]
                hm = lax.rem(h, NUM_BUCKETS)
                hbuf[b][pl.ds(i * LANES, LANES)] = hm
                pbuf[b][pl.ds(i * LANES, LANES)] = lax.shift_right_logical(hm, 1)

        def g_fire(b):
            pltpu.async_copy(scr.at[pbuf[b]], gbuf[b], gsem[b])

        def g_wait(b):
            pltpu.make_async_copy(scr.at[pbuf[b]], gbuf[b], gsem[b]).wait()

        def transpose_unit(b, o):
            ob = obuf[o]

            @plsc.parallel_loop(0, 32, unroll=4)
            def blk(bi):
                jbi = lax.shift_right_logical(bi, 2)
                dgi = bi & 3
                jsrc = iota + jbi * LANES
                parv = (hbuf[b][pl.ds(jbi * LANES, LANES)] & 1) * EMBED_DIM
                d0 = dgi * LANES
                for t in range(LANES):
                    perm = (iota + t) & 15
                    dsrc = perm + d0
                    val = plsc.load_gather(gbuf[b], [jsrc, parv + dsrc])
                    plsc.store_scatter(ob, [dsrc, jsrc], val)

        def o_write(k, b):
            u = u0 + k
            s = lax.shift_right_logical(u, 7)
            tb = u & 127
            pltpu.async_copy(
                obuf[b], out.at[s, :, pl.ds(tb * 128, 128)], osem[b]
            )

        def o_wait(b):
            pltpu.make_async_copy(
                obuf[b], out.at[0, :, pl.ds(0, 128)], osem[b]
            ).wait()

        for kk in range(3):
            idx_load(kk, kk)
        for kk in range(3):
            idx_wait(kk)
            mod(kk)
            g_fire(kk)
        idx_load(3, 3)

        def quad_body(k4, carry):
            for b in range(4):
                k = k4 * 4 + b
                ob = b % 2

                @pl.when(k + 3 < n_units)
                def _():
                    nb = (b + 3) % 4
                    idx_wait(nb)
                    mod(nb)
                    g_fire(nb)

                    @pl.when(k + 4 < n_units)
                    def _():
                        idx_load(k + 4, b)

                g_wait(b)

                @pl.when(k >= 2)
                def _():
                    o_wait(ob)

                transpose_unit(b, ob)
                o_write(k, ob)
            return carry

        lax.fori_loop(0, n_units // 4, quad_body, 0)
        o_wait(0)
        o_wait(1)

    return lookup


def kernel(feature_values, table):
    table_t = table.T
    fv_t = feature_values.T
    tail = table[FULL_TILES * 128:].reshape(32, 2 * EMBED_DIM)
    scr = _make_repack()(table_t, tail)
    out_phys = _make_lookup()(scr, fv_t)
    return jnp.transpose(out_phys, (2, 0, 1))


# speedup vs baseline: 1.0956x; 1.0956x over previous
import functools

import jax
import jax.numpy as jnp
from jax import lax
from jax.experimental import pallas as pl
from jax.experimental.pallas import tpu as pltpu
from jax.experimental.pallas import tpu_sc as plsc

NUM_BUCKETS = 1000000
EMBED_DIM = 64
LANES = 16
BATCH = 16384
SEQ = 50

PAIR_ROWS = 500032
FULL_TILES = 7812
MAIN_TILES = 7808
TILES_PER_W = 244
UNITS_PER_W = 200


def _mesh():
    return plsc.VectorSubcoreMesh(core_axis_name="c", subcore_axis_name="s")


def _wid(nc):
    return lax.axis_index("s") * nc + lax.axis_index("c")


@functools.lru_cache(maxsize=None)
def _make_repack():
    info = plsc.get_sparse_core_info()
    nc = info.num_cores

    @functools.partial(
        pl.kernel,
        mesh=_mesh(),
        out_type=jax.ShapeDtypeStruct((PAIR_ROWS, 2 * EMBED_DIM), jnp.float32),
        compiler_params=pltpu.CompilerParams(
            use_tc_tiling_on_sc=True,
            needs_layout_passes=False,
            disable_bounds_checks=True,
        ),
        scratch_types=[
            [pltpu.VMEM((EMBED_DIM, 128), jnp.float32) for _ in range(2)],
            [pltpu.VMEM((EMBED_DIM, 2 * EMBED_DIM), jnp.float32) for _ in range(2)],
            [pltpu.SemaphoreType.DMA for _ in range(2)],
            [pltpu.SemaphoreType.DMA for _ in range(2)],
        ],
    )
    def repack(table_t, tail, scr, bin_, bouts, rsem, wsem):
        wid = _wid(nc)
        iota = lax.iota(jnp.int32, LANES)

        def read(ti, b):
            pltpu.async_copy(
                table_t.at[:, pl.ds(ti * 128, 128)], bin_[b], rsem[b]
            )

        def wait_read(b):
            pltpu.make_async_copy(
                table_t.at[:, pl.ds(0, 128)], bin_[b], rsem[b]
            ).wait()

        def transpose(b, n_buckets):
            bout = bouts[b]
            n_blocks = 4 * (n_buckets // LANES)

            @plsc.parallel_loop(0, n_blocks, unroll=4)
            def blk(bi):
                dgi = lax.shift_right_logical(bi, 3)
                jbi = bi & 7
                dsrc = iota + dgi * LANES
                j0 = jbi * LANES
                for t in range(LANES):
                    perm = (iota + t) & 15
                    jsrc = perm + j0
                    val = plsc.load_gather(bin_[b], [dsrc, jsrc])
                    rowdst = lax.shift_right_logical(jsrc, 1)
                    coldst = (jsrc & 1) * EMBED_DIM + dsrc
                    plsc.store_scatter(bout, [rowdst, coldst], val)

        def write(ti, b):
            pltpu.async_copy(bouts[b], scr.at[pl.ds(ti * 64, 64), :], wsem[b])

        def wait_write(b):
            pltpu.make_async_copy(
                bouts[b], scr.at[pl.ds(0, 64), :], wsem[b]
            ).wait()

        read(wid, 0)

        def pair_body(k2, carry):
            for b in range(2):
                k = k2 * 2 + b
                ti = wid + k * 32

                @pl.when(k + 1 < TILES_PER_W)
                def _():
                    read(wid + (k + 1) * 32, 1 - b)

                wait_read(b)

                @pl.when(k >= 2)
                def _():
                    wait_write(b)

                transpose(b, 128)
                write(ti, b)
            return carry

        lax.fori_loop(0, TILES_PER_W // 2, pair_body, 0)
        wait_write(0)
        wait_write(1)

        @pl.when(wid < FULL_TILES - MAIN_TILES)
        def _():
            ti = MAIN_TILES + wid
            pltpu.sync_copy(table_t.at[:, pl.ds(ti * 128, 128)], bin_[0])
            transpose(0, 128)
            write(ti, 0)
            wait_write(0)

        @pl.when(wid == 4)
        def _():
            pltpu.sync_copy(tail, scr.at[pl.ds(FULL_TILES * 64, 32), :])

    return repack


@functools.lru_cache(maxsize=None)
def _make_lookup():
    info = plsc.get_sparse_core_info()
    nc = info.num_cores
    n_units = UNITS_PER_W

    @functools.partial(
        pl.kernel,
        mesh=_mesh(),
        out_type=jax.ShapeDtypeStruct((SEQ, EMBED_DIM, BATCH), jnp.float32),
        compiler_params=pltpu.CompilerParams(
            use_tc_tiling_on_sc=True,
            needs_layout_passes=False,
            disable_bounds_checks=True,
        ),
        scratch_types=[
            [pltpu.VMEM((128,), jnp.int32) for _ in range(2)],
            [pltpu.VMEM((128,), jnp.int32) for _ in range(2)],
            [pltpu.VMEM((128,), jnp.int32) for _ in range(2)],
            [pltpu.VMEM((128, 2 * EMBED_DIM), jnp.float32) for _ in range(2)],
            [pltpu.VMEM((EMBED_DIM, 128), jnp.float32) for _ in range(2)],
            [pltpu.SemaphoreType.DMA for _ in range(2)],
            [pltpu.SemaphoreType.DMA for _ in range(2)],
            [pltpu.SemaphoreType.DMA for _ in range(2)],
        ],
    )
    def lookup(scr, fv_t, out, ibuf, hbuf, pbuf, gbuf, obuf, isem, gsem, osem):
        wid = _wid(nc)
        u0 = wid * n_units
        iota = lax.iota(jnp.int32, LANES)

        def idx_load(k, b):
            u = u0 + k
            s = lax.shift_right_logical(u, 7)
            tb = u & 127
            pltpu.async_copy(
                fv_t.at[s, pl.ds(tb * 128, 128)], ibuf[b], isem[b]
            )

        def idx_wait(b):
            pltpu.make_async_copy(
                fv_t.at[0, pl.ds(0, 128)], ibuf[b], isem[b]
            ).wait()

        def mod(b):
            for i in range(8):
                h = ibuf[b][pl.ds(i * LANES, LANES)]
                hm = lax.rem(h, NUM_BUCKETS)
                hbuf[b][pl.ds(i * LANES, LANES)] = hm
                pbuf[b][pl.ds(i * LANES, LANES)] = lax.shift_right_logical(hm, 1)

        def g_fire(b):
            pltpu.async_copy(scr.at[pbuf[b]], gbuf[b], gsem[b])

        def g_wait(b):
            pltpu.make_async_copy(scr.at[pbuf[b]], gbuf[b], gsem[b]).wait()

        def transpose_unit(b, o):
            ob = obuf[o]

            @plsc.parallel_loop(0, 32, unroll=4)
            def blk(bi):
                jbi = lax.shift_right_logical(bi, 2)
                dgi = bi & 3
                jsrc = iota + jbi * LANES
                parv = (hbuf[b][pl.ds(jbi * LANES, LANES)] & 1) * EMBED_DIM
                d0 = dgi * LANES
                for t in range(LANES):
                    perm = (iota + t) & 15
                    dsrc = perm + d0
                    val = plsc.load_gather(gbuf[b], [jsrc, parv + dsrc])
                    plsc.store_scatter(ob, [dsrc, jsrc], val)

        def o_write(k, b):
            u = u0 + k
            s = lax.shift_right_logical(u, 7)
            tb = u & 127
            pltpu.async_copy(
                obuf[b], out.at[s, :, pl.ds(tb * 128, 128)], osem[b]
            )

        def o_wait(b):
            pltpu.make_async_copy(
                obuf[b], out.at[0, :, pl.ds(0, 128)], osem[b]
            ).wait()

        idx_load(0, 0)
        idx_wait(0)
        mod(0)
        idx_load(1, 1)
        g_fire(0)

        def pair_body(k2, carry):
            for b in range(2):
                k = k2 * 2 + b

                @pl.when(k < n_units - 1)
                def _():
                    idx_wait(1 - b)
                    mod(1 - b)

                    @pl.when(k < n_units - 2)
                    def _():
                        idx_load(k + 2, b)

                    g_fire(1 - b)

                g_wait(b)

                @pl.when(k >= 2)
                def _():
                    o_wait(b)

                transpose_unit(b, b)
                o_write(k, b)
            return carry

        lax.fori_loop(0, n_units // 2, pair_body, 0)
        o_wait(0)
        o_wait(1)

    return lookup


def kernel(feature_values, table):
    table_t = table.T
    fv_t = feature_values.T
    tail = table[FULL_TILES * 128:].reshape(32, 2 * EMBED_DIM)
    scr = _make_repack()(table_t, tail)
    out_phys = _make_lookup()(scr, fv_t)
    return jnp.transpose(out_phys, (2, 0, 1))
